# Initial kernel scaffold; baseline (speedup 1.0000x reference)
import jax
import jax.numpy as jnp
from jax.experimental import pallas as pl


def kernel(x, batch, W1, b1, W2, b2):
    raise NotImplementedError("write your pallas kernel here")


# speedup vs baseline: 4.8317x; 4.8317x over previous
import jax
import jax.numpy as jnp
from jax import lax
from jax.experimental import pallas as pl
from jax.experimental.pallas import tpu as pltpu

G = 256
BN = 1000


def _gate_body(x_ref, w1t_ref, b1_ref, w2_ref, b2_ref, g_ref, gmax_ref):
    x = x_ref[...]
    pre = jnp.dot(x, w1t_ref[...], preferred_element_type=jnp.float32)
    h = jnp.tanh(pre + b1_ref[...])
    g = jnp.sum(h * w2_ref[...], axis=1) + b2_ref[0]
    g_ref[0, 0, :] = g

    @pl.when(pl.program_id(0) == 0)
    def _():
        gmax_ref[0] = -jnp.inf
    gmax_ref[0] = jnp.maximum(gmax_ref[0], jnp.max(g))


def _denom_body(g_ref, b_ref, gmax_ref, denom_ref):
    @pl.when(pl.program_id(0) == 0)
    def _():
        denom_ref[...] = jnp.zeros_like(denom_ref)
    e = jnp.exp(g_ref[0, 0, :] - gmax_ref[0])
    seg = b_ref[0, 0, :]
    oh = (seg[:, None] == lax.broadcasted_iota(jnp.int32, (BN, G), 1))
    oh = oh.astype(jnp.float32)
    denom_ref[...] += jnp.dot(e[None, :], oh, preferred_element_type=jnp.float32)


def _pool_body(x_ref, b_ref, g_ref, gmax_ref, denom_ref,
               attn_ref, mean_ref, cnt_ref):
    i = pl.program_id(0)

    @pl.when(i == 0)
    def _():
        attn_ref[...] = jnp.zeros_like(attn_ref)
        mean_ref[...] = jnp.zeros_like(mean_ref)
        cnt_ref[...] = jnp.zeros_like(cnt_ref)

    x = x_ref[...]
    seg = b_ref[0, 0, :]
    oh = (lax.broadcasted_iota(jnp.int32, (G, BN), 0) == seg[None, :])
    oh = oh.astype(jnp.float32)
    e = jnp.exp(g_ref[0, 0, :] - gmax_ref[0])
    drow = jnp.dot(denom_ref[...], oh,
                   preferred_element_type=jnp.float32)
    w = e / (drow[0, :] + 1e-8)
    attn_ref[...] += jnp.dot(oh, w[:, None] * x,
                             preferred_element_type=jnp.float32)
    mean_ref[...] += jnp.dot(oh, x, preferred_element_type=jnp.float32)
    cnt_ref[...] += jnp.sum(oh, axis=1)[:, None]

    @pl.when(i == pl.num_programs(0) - 1)
    def _():
        c = cnt_ref[...]
        mean_ref[...] = mean_ref[...] / jnp.where(c > 0.0, c, 1.0)


def kernel(x, batch, W1, b1, W2, b2):
    n, d = x.shape
    h = W1.shape[0]
    nblk = n // BN
    w1t = W1.T
    b1r = b1.reshape(1, h)
    w2r = W2.reshape(1, h)
    b2r = b2.reshape(1)
    batch3 = batch.reshape(nblk, 1, BN)

    g3, gmax = pl.pallas_call(
        _gate_body,
        grid=(nblk,),
        in_specs=[
            pl.BlockSpec((BN, d), lambda i: (i, 0)),
            pl.BlockSpec((d, h), lambda i: (0, 0)),
            pl.BlockSpec((1, h), lambda i: (0, 0)),
            pl.BlockSpec((1, h), lambda i: (0, 0)),
            pl.BlockSpec(memory_space=pltpu.SMEM),
        ],
        out_specs=[
            pl.BlockSpec((1, 1, BN), lambda i: (i, 0, 0)),
            pl.BlockSpec(memory_space=pltpu.SMEM),
        ],
        out_shape=[
            jax.ShapeDtypeStruct((nblk, 1, BN), jnp.float32),
            jax.ShapeDtypeStruct((1,), jnp.float32),
        ],
    )(x, w1t, b1r, w2r, b2r)

    denom = pl.pallas_call(
        _denom_body,
        grid=(nblk,),
        in_specs=[
            pl.BlockSpec((1, 1, BN), lambda i: (i, 0, 0)),
            pl.BlockSpec((1, 1, BN), lambda i: (i, 0, 0)),
            pl.BlockSpec(memory_space=pltpu.SMEM),
        ],
        out_specs=pl.BlockSpec((1, G), lambda i: (0, 0)),
        out_shape=jax.ShapeDtypeStruct((1, G), jnp.float32),
    )(g3, batch3, gmax)

    attn, mean, _cnt = pl.pallas_call(
        _pool_body,
        grid=(nblk,),
        in_specs=[
            pl.BlockSpec((BN, d), lambda i: (i, 0)),
            pl.BlockSpec((1, 1, BN), lambda i: (i, 0, 0)),
            pl.BlockSpec((1, 1, BN), lambda i: (i, 0, 0)),
            pl.BlockSpec(memory_space=pltpu.SMEM),
            pl.BlockSpec((1, G), lambda i: (0, 0)),
        ],
        out_specs=[
            pl.BlockSpec((G, d), lambda i: (0, 0)),
            pl.BlockSpec((G, d), lambda i: (0, 0)),
            pl.BlockSpec((G, 1), lambda i: (0, 0)),
        ],
        out_shape=[
            jax.ShapeDtypeStruct((G, d), jnp.float32),
            jax.ShapeDtypeStruct((G, d), jnp.float32),
            jax.ShapeDtypeStruct((G, 1), jnp.float32),
        ],
    )(x, batch3, g3, gmax, denom)

    return jnp.concatenate([attn, mean], axis=-1)


# speedup vs baseline: 4.8349x; 1.0007x over previous
import jax
import jax.numpy as jnp
from jax import lax
from jax.experimental import pallas as pl
from jax.experimental.pallas import tpu as pltpu

G = 256
BN = 5000
BND = 10000


def _gate_body(x_ref, w1t_ref, b1_ref, w2_ref, b2_ref, g_ref, gmax_ref):
    x = x_ref[...]
    pre = jnp.dot(x, w1t_ref[...], preferred_element_type=jnp.float32)
    h = jnp.tanh(pre + b1_ref[...])
    g = jnp.sum(h * w2_ref[...], axis=1) + b2_ref[0]
    g_ref[0, 0, :] = g

    @pl.when(pl.program_id(0) == 0)
    def _():
        gmax_ref[0] = -jnp.inf
    gmax_ref[0] = jnp.maximum(gmax_ref[0], jnp.max(g))


def _denom_body(g_ref, b_ref, gmax_ref, denom_ref):
    @pl.when(pl.program_id(0) == 0)
    def _():
        denom_ref[...] = jnp.zeros_like(denom_ref)
    e = jnp.exp(g_ref[0, 0, :] - gmax_ref[0])
    seg = b_ref[0, 0, :]
    oh = (seg[:, None] == lax.broadcasted_iota(jnp.int32, (BND, G), 1))
    denom_ref[...] += jnp.dot(e[None, :].astype(jnp.bfloat16),
                              oh.astype(jnp.bfloat16),
                              preferred_element_type=jnp.float32)


def _pool_body(x_ref, b_ref, g_ref, gmax_ref, denom_ref,
               attn_ref, mean_ref, cnt_ref):
    i = pl.program_id(0)

    @pl.when(i == 0)
    def _():
        attn_ref[...] = jnp.zeros_like(attn_ref)
        mean_ref[...] = jnp.zeros_like(mean_ref)
        cnt_ref[...] = jnp.zeros_like(cnt_ref)

    x = x_ref[...]
    seg = b_ref[0, 0, :]
    ohb = (lax.broadcasted_iota(jnp.int32, (G, BN), 0) == seg[None, :])
    oh = ohb.astype(jnp.bfloat16)
    e = jnp.exp(g_ref[0, 0, :] - gmax_ref[0])
    drow = jnp.dot(denom_ref[...], ohb.astype(jnp.float32),
                   preferred_element_type=jnp.float32)
    w = e / (drow[0, :] + 1e-8)
    wx = (w[:, None] * x.astype(jnp.float32)).astype(jnp.bfloat16)
    attn_ref[...] += jnp.dot(oh, wx, preferred_element_type=jnp.float32)
    mean_ref[...] += jnp.dot(oh, x, preferred_element_type=jnp.float32)
    cnt_ref[...] += jnp.sum(ohb.astype(jnp.float32), axis=1)[:, None]

    @pl.when(i == pl.num_programs(0) - 1)
    def _():
        c = cnt_ref[...]
        mean_ref[...] = mean_ref[...] / jnp.where(c > 0.0, c, 1.0)


def kernel(x, batch, W1, b1, W2, b2):
    n, d = x.shape
    h = W1.shape[0]
    nblk = n // BN
    nblkd = n // BND
    xbf = x.astype(jnp.bfloat16)
    w1t = W1.T.astype(jnp.bfloat16)
    b1r = b1.reshape(1, h)
    w2r = W2.reshape(1, h)
    b2r = b2.reshape(1)
    batch3 = batch.reshape(nblk, 1, BN)
    batch3d = batch.reshape(nblkd, 1, BND)

    g3, gmax = pl.pallas_call(
        _gate_body,
        grid=(nblk,),
        in_specs=[
            pl.BlockSpec((BN, d), lambda i: (i, 0)),
            pl.BlockSpec((d, h), lambda i: (0, 0)),
            pl.BlockSpec((1, h), lambda i: (0, 0)),
            pl.BlockSpec((1, h), lambda i: (0, 0)),
            pl.BlockSpec(memory_space=pltpu.SMEM),
        ],
        out_specs=[
            pl.BlockSpec((1, 1, BN), lambda i: (i, 0, 0)),
            pl.BlockSpec(memory_space=pltpu.SMEM),
        ],
        out_shape=[
            jax.ShapeDtypeStruct((nblk, 1, BN), jnp.float32),
            jax.ShapeDtypeStruct((1,), jnp.float32),
        ],
    )(xbf, w1t, b1r, w2r, b2r)

    g3d = g3.reshape(nblkd, 1, BND)

    denom = pl.pallas_call(
        _denom_body,
        grid=(nblkd,),
        in_specs=[
            pl.BlockSpec((1, 1, BND), lambda i: (i, 0, 0)),
            pl.BlockSpec((1, 1, BND), lambda i: (i, 0, 0)),
            pl.BlockSpec(memory_space=pltpu.SMEM),
        ],
        out_specs=pl.BlockSpec((1, G), lambda i: (0, 0)),
        out_shape=jax.ShapeDtypeStruct((1, G), jnp.float32),
    )(g3d, batch3d, gmax)

    attn, mean, _cnt = pl.pallas_call(
        _pool_body,
        grid=(nblk,),
        in_specs=[
            pl.BlockSpec((BN, d), lambda i: (i, 0)),
            pl.BlockSpec((1, 1, BN), lambda i: (i, 0, 0)),
            pl.BlockSpec((1, 1, BN), lambda i: (i, 0, 0)),
            pl.BlockSpec(memory_space=pltpu.SMEM),
            pl.BlockSpec((1, G), lambda i: (0, 0)),
        ],
        out_specs=[
            pl.BlockSpec((G, d), lambda i: (0, 0)),
            pl.BlockSpec((G, d), lambda i: (0, 0)),
            pl.BlockSpec((G, 1), lambda i: (0, 0)),
        ],
        out_shape=[
            jax.ShapeDtypeStruct((G, d), jnp.float32),
            jax.ShapeDtypeStruct((G, d), jnp.float32),
            jax.ShapeDtypeStruct((G, 1), jnp.float32),
        ],
    )(xbf, batch3, g3, gmax, denom)

    return jnp.concatenate([attn, mean], axis=-1)


# speedup vs baseline: 9.8066x; 2.0283x over previous
import jax
import jax.numpy as jnp
from jax import lax
from jax.experimental import pallas as pl
from jax.experimental.pallas import tpu as pltpu

G = 256
BN = 5000
NEG = -1e30


def _body(x_ref, seg_ref, w1t_ref, b1_ref, w2_ref, b2_ref,
          attn_ref, mean_ref,
          m_ref, d_ref, cnt_ref, gmax_ref):
    i = pl.program_id(0)

    @pl.when(i == 0)
    def _():
        attn_ref[...] = jnp.zeros_like(attn_ref)
        mean_ref[...] = jnp.zeros_like(mean_ref)
        cnt_ref[...] = jnp.zeros_like(cnt_ref)
        m_ref[...] = jnp.full_like(m_ref, NEG)
        d_ref[...] = jnp.zeros_like(d_ref)
        gmax_ref[0] = NEG

    x = x_ref[...]
    xb = x.astype(jnp.bfloat16)

    pre = jnp.dot(xb, w1t_ref[...], preferred_element_type=jnp.float32)
    h = jnp.tanh(pre + b1_ref[...])
    g_row = lax.dot_general(w2_ref[...], h, (((1,), (1,)), ((), ())),
                            preferred_element_type=jnp.float32)
    g_row = g_row + b2_ref[0]
    gmax_ref[0] = jnp.maximum(gmax_ref[0], jnp.max(g_row))

    seg = seg_ref[0, 0, :]
    ohb = (lax.broadcasted_iota(jnp.int32, (G, BN), 0) == seg[None, :])
    oh_bf = ohb.astype(jnp.bfloat16)
    oh_f = ohb.astype(jnp.float32)

    gb = jnp.broadcast_to(g_row, (G, BN))
    m_blk = jnp.max(jnp.where(ohb, gb, NEG), axis=1)[None, :]

    m_old = m_ref[...]
    m_new = jnp.maximum(m_old, m_blk)
    scale = jnp.exp(m_old - m_new)
    m_ref[...] = m_new

    m_row = jnp.dot(m_new, oh_f, preferred_element_type=jnp.float32)
    e_row = jnp.exp(g_row - m_row)

    s_blk = lax.dot_general(e_row, oh_f, (((1,), (1,)), ((), ())),
                            preferred_element_type=jnp.float32)
    d_ref[...] = d_ref[...] * scale + s_blk

    scale_col = scale.reshape(G, 1)
    ew = oh_bf * e_row.astype(jnp.bfloat16)
    attn_ref[...] = attn_ref[...] * scale_col + jnp.dot(
        ew, xb, preferred_element_type=jnp.float32)
    mean_ref[...] += jnp.dot(oh_bf, xb, preferred_element_type=jnp.float32)
    ones_col = jnp.ones((BN, 1), dtype=jnp.bfloat16)
    cnt_ref[...] += jnp.dot(oh_bf, ones_col, preferred_element_type=jnp.float32)

    @pl.when(i == pl.num_programs(0) - 1)
    def _():
        m_col = m_ref[...].reshape(G, 1)
        d_col = d_ref[...].reshape(G, 1)
        eps = 1e-8 * jnp.exp(gmax_ref[0] - m_col)
        attn_ref[...] = attn_ref[...] / (d_col + eps)
        c = cnt_ref[...]
        mean_ref[...] = mean_ref[...] / jnp.where(c > 0.0, c, 1.0)


def kernel(x, batch, W1, b1, W2, b2):
    n, d = x.shape
    hd = W1.shape[0]
    nblk = n // BN
    w1t = W1.T.astype(jnp.bfloat16)
    b1r = b1.reshape(1, hd)
    w2r = W2.reshape(1, hd)
    b2r = b2.reshape(1)
    batch3 = batch.reshape(nblk, 1, BN)

    attn, mean = pl.pallas_call(
        _body,
        grid=(nblk,),
        in_specs=[
            pl.BlockSpec((BN, d), lambda i: (i, 0)),
            pl.BlockSpec((1, 1, BN), lambda i: (i, 0, 0)),
            pl.BlockSpec((d, hd), lambda i: (0, 0)),
            pl.BlockSpec((1, hd), lambda i: (0, 0)),
            pl.BlockSpec((1, hd), lambda i: (0, 0)),
            pl.BlockSpec(memory_space=pltpu.SMEM),
        ],
        out_specs=[
            pl.BlockSpec((G, d), lambda i: (0, 0)),
            pl.BlockSpec((G, d), lambda i: (0, 0)),
        ],
        out_shape=[
            jax.ShapeDtypeStruct((G, d), jnp.float32),
            jax.ShapeDtypeStruct((G, d), jnp.float32),
        ],
        scratch_shapes=[
            pltpu.VMEM((1, G), jnp.float32),
            pltpu.VMEM((1, G), jnp.float32),
            pltpu.VMEM((G, 1), jnp.float32),
            pltpu.SMEM((1,), jnp.float32),
        ],
    )(x, batch3, w1t, b1r, w2r, b2r)

    return jnp.concatenate([attn, mean], axis=-1)


# speedup vs baseline: 15.3140x; 1.5616x over previous
import jax
import jax.numpy as jnp
from jax import lax
from jax.experimental import pallas as pl
from jax.experimental.pallas import tpu as pltpu

G = 256
BN = 5000
NCH = 4
GB = G // NCH
NEG = -1e30


def _body(x_ref, seg_ref, w1t_ref, b1_ref, w2_ref, b2_ref,
          attn_ref, mean_ref,
          m_ref, d_ref, cnt_ref, gmax_ref):
    i = pl.program_id(0)

    @pl.when(i == 0)
    def _():
        attn_ref[...] = jnp.zeros_like(attn_ref)
        mean_ref[...] = jnp.zeros_like(mean_ref)
        cnt_ref[...] = jnp.zeros_like(cnt_ref)
        m_ref[...] = jnp.full_like(m_ref, NEG)
        d_ref[...] = jnp.zeros_like(d_ref)
        gmax_ref[0] = NEG

    x = x_ref[...]
    xb = x.astype(jnp.bfloat16)

    pre = jnp.dot(xb, w1t_ref[...], preferred_element_type=jnp.float32)
    h = jnp.tanh(pre + b1_ref[...])
    g_row = lax.dot_general(w2_ref[...], h, (((1,), (1,)), ((), ())),
                            preferred_element_type=jnp.float32)
    g_row = g_row + b2_ref[0]
    gmax_ref[0] = jnp.maximum(gmax_ref[0], jnp.max(g_row))

    seg = seg_ref[0, 0, :]
    smin = jnp.min(seg)
    smax = jnp.max(seg)
    ones_col = jnp.ones((BN, 1), dtype=jnp.bfloat16)

    for c in range(NCH):
        lo = c * GB

        @pl.when((smax >= lo) & (smin < lo + GB))
        def _(lo=lo):
            in_chunk = (seg >= lo) & (seg < lo + GB)
            ohb = (lax.broadcasted_iota(jnp.int32, (GB, BN), 0) + lo
                   == seg[None, :])
            oh_bf = ohb.astype(jnp.bfloat16)

            gbc = jnp.broadcast_to(g_row, (GB, BN))
            m_blk = jnp.max(jnp.where(ohb, gbc, NEG), axis=1)[None, :]
            m_bf = m_blk.astype(jnp.bfloat16)
            m_sh = m_bf.astype(jnp.float32)

            m_old = m_ref[:, lo:lo + GB]
            m_new = jnp.maximum(m_old, m_blk)
            scale = jnp.exp(m_old - m_new)
            corr = jnp.exp(m_sh - m_new)
            m_ref[:, lo:lo + GB] = m_new

            m_row = jnp.dot(m_bf, oh_bf,
                            preferred_element_type=jnp.float32)
            e_row = jnp.exp(jnp.where(in_chunk[None, :],
                                      g_row - m_row, NEG))
            e_bf = e_row.astype(jnp.bfloat16)

            s_blk = lax.dot_general(e_bf, oh_bf, (((1,), (1,)), ((), ())),
                                    preferred_element_type=jnp.float32)
            d_ref[:, lo:lo + GB] = (d_ref[:, lo:lo + GB] * scale
                                    + s_blk * corr)

            ew = oh_bf * e_bf
            scale_col = scale.reshape(GB, 1)
            corr_col = corr.reshape(GB, 1)
            attn_ref[lo:lo + GB, :] = (
                attn_ref[lo:lo + GB, :] * scale_col
                + jnp.dot(ew, xb, preferred_element_type=jnp.float32)
                * corr_col)
            mean_ref[lo:lo + GB, :] += jnp.dot(
                oh_bf, xb, preferred_element_type=jnp.float32)
            cnt_ref[lo:lo + GB, :] += jnp.dot(
                oh_bf, ones_col, preferred_element_type=jnp.float32)

    @pl.when(i == pl.num_programs(0) - 1)
    def _():
        m_col = m_ref[...].reshape(G, 1)
        d_col = d_ref[...].reshape(G, 1)
        eps = 1e-8 * jnp.exp(gmax_ref[0] - m_col)
        attn_ref[...] = attn_ref[...] / (d_col + eps)
        c = cnt_ref[...]
        mean_ref[...] = mean_ref[...] / jnp.where(c > 0.0, c, 1.0)


def kernel(x, batch, W1, b1, W2, b2):
    n, d = x.shape
    hd = W1.shape[0]
    nblk = n // BN
    w1t = W1.T.astype(jnp.bfloat16)
    b1r = b1.reshape(1, hd)
    w2r = W2.reshape(1, hd)
    b2r = b2.reshape(1)
    batch3 = batch.reshape(nblk, 1, BN)

    attn, mean = pl.pallas_call(
        _body,
        grid=(nblk,),
        in_specs=[
            pl.BlockSpec((BN, d), lambda i: (i, 0)),
            pl.BlockSpec((1, 1, BN), lambda i: (i, 0, 0)),
            pl.BlockSpec((d, hd), lambda i: (0, 0)),
            pl.BlockSpec((1, hd), lambda i: (0, 0)),
            pl.BlockSpec((1, hd), lambda i: (0, 0)),
            pl.BlockSpec(memory_space=pltpu.SMEM),
        ],
        out_specs=[
            pl.BlockSpec((G, d), lambda i: (0, 0)),
            pl.BlockSpec((G, d), lambda i: (0, 0)),
        ],
        out_shape=[
            jax.ShapeDtypeStruct((G, d), jnp.float32),
            jax.ShapeDtypeStruct((G, d), jnp.float32),
        ],
        scratch_shapes=[
            pltpu.VMEM((1, G), jnp.float32),
            pltpu.VMEM((1, G), jnp.float32),
            pltpu.VMEM((G, 1), jnp.float32),
            pltpu.SMEM((1,), jnp.float32),
        ],
    )(x, batch3, w1t, b1r, w2r, b2r)

    return jnp.concatenate([attn, mean], axis=-1)


# speedup vs baseline: 18.5483x; 1.2112x over previous
import jax
import jax.numpy as jnp
from jax import lax
from jax.experimental import pallas as pl
from jax.experimental.pallas import tpu as pltpu

G = 256
BN = 10000
NCH = 4
GB = G // NCH
NEG = -1e30


def _body(x_ref, seg_ref, w1t_ref, b1_ref, w2_ref, b2_ref,
          attn_ref, mean_ref,
          m_ref, d_ref, cnt_ref, gmax_ref):
    i = pl.program_id(0)

    @pl.when(i == 0)
    def _():
        attn_ref[...] = jnp.zeros_like(attn_ref)
        mean_ref[...] = jnp.zeros_like(mean_ref)
        cnt_ref[...] = jnp.zeros_like(cnt_ref)
        m_ref[...] = jnp.full_like(m_ref, NEG)
        d_ref[...] = jnp.zeros_like(d_ref)
        gmax_ref[0] = NEG

    x = x_ref[...]
    xb = x.astype(jnp.bfloat16)

    pre = jnp.dot(xb, w1t_ref[...], preferred_element_type=jnp.float32)
    h = jnp.tanh(pre + b1_ref[...])
    g_row = lax.dot_general(w2_ref[...], h, (((1,), (1,)), ((), ())),
                            preferred_element_type=jnp.float32)
    g_row = g_row + b2_ref[0]
    gmax_ref[0] = jnp.maximum(gmax_ref[0], jnp.max(g_row))

    seg = seg_ref[0, 0, :]
    smin = jnp.min(seg)
    smax = jnp.max(seg)
    ones_col = jnp.ones((BN, 1), dtype=jnp.bfloat16)

    for c in range(NCH):
        lo = c * GB

        @pl.when((smax >= lo) & (smin < lo + GB))
        def _(lo=lo):
            in_chunk = (seg >= lo) & (seg < lo + GB)
            ohb = (lax.broadcasted_iota(jnp.int32, (GB, BN), 0) + lo
                   == seg[None, :])
            oh_bf = ohb.astype(jnp.bfloat16)

            gbc = jnp.broadcast_to(g_row, (GB, BN))
            m_blk = jnp.max(jnp.where(ohb, gbc, NEG), axis=1)[None, :]
            m_bf = m_blk.astype(jnp.bfloat16)
            m_sh = m_bf.astype(jnp.float32)

            m_old = m_ref[:, lo:lo + GB]
            m_new = jnp.maximum(m_old, m_blk)
            scale = jnp.exp(m_old - m_new)
            corr = jnp.exp(m_sh - m_new)
            m_ref[:, lo:lo + GB] = m_new

            m_row = jnp.dot(m_bf, oh_bf,
                            preferred_element_type=jnp.float32)
            e_row = jnp.exp(jnp.where(in_chunk[None, :],
                                      g_row - m_row, NEG))
            e_bf = e_row.astype(jnp.bfloat16)

            s_blk = lax.dot_general(e_bf, oh_bf, (((1,), (1,)), ((), ())),
                                    preferred_element_type=jnp.float32)
            d_ref[:, lo:lo + GB] = (d_ref[:, lo:lo + GB] * scale
                                    + s_blk * corr)

            ew2 = jnp.concatenate([oh_bf * e_bf, oh_bf], axis=0)
            out = jnp.dot(ew2, xb, preferred_element_type=jnp.float32)
            scale_col = scale.reshape(GB, 1)
            corr_col = corr.reshape(GB, 1)
            attn_ref[lo:lo + GB, :] = (
                attn_ref[lo:lo + GB, :] * scale_col
                + out[:GB, :] * corr_col)
            mean_ref[lo:lo + GB, :] += out[GB:, :]
            cnt_ref[lo:lo + GB, :] += jnp.dot(
                oh_bf, ones_col, preferred_element_type=jnp.float32)

    @pl.when(i == pl.num_programs(0) - 1)
    def _():
        m_col = m_ref[...].reshape(G, 1)
        d_col = d_ref[...].reshape(G, 1)
        eps = 1e-8 * jnp.exp(gmax_ref[0] - m_col)
        attn_ref[...] = attn_ref[...] / (d_col + eps)
        c = cnt_ref[...]
        mean_ref[...] = mean_ref[...] / jnp.where(c > 0.0, c, 1.0)


def kernel(x, batch, W1, b1, W2, b2):
    n, d = x.shape
    hd = W1.shape[0]
    nblk = n // BN
    w1t = W1.T.astype(jnp.bfloat16)
    b1r = b1.reshape(1, hd)
    w2r = W2.reshape(1, hd)
    b2r = b2.reshape(1)
    batch3 = batch.reshape(nblk, 1, BN)

    attn, mean = pl.pallas_call(
        _body,
        grid=(nblk,),
        in_specs=[
            pl.BlockSpec((BN, d), lambda i: (i, 0)),
            pl.BlockSpec((1, 1, BN), lambda i: (i, 0, 0)),
            pl.BlockSpec((d, hd), lambda i: (0, 0)),
            pl.BlockSpec((1, hd), lambda i: (0, 0)),
            pl.BlockSpec((1, hd), lambda i: (0, 0)),
            pl.BlockSpec(memory_space=pltpu.SMEM),
        ],
        out_specs=[
            pl.BlockSpec((G, d), lambda i: (0, 0)),
            pl.BlockSpec((G, d), lambda i: (0, 0)),
        ],
        out_shape=[
            jax.ShapeDtypeStruct((G, d), jnp.float32),
            jax.ShapeDtypeStruct((G, d), jnp.float32),
        ],
        scratch_shapes=[
            pltpu.VMEM((1, G), jnp.float32),
            pltpu.VMEM((1, G), jnp.float32),
            pltpu.VMEM((G, 1), jnp.float32),
            pltpu.SMEM((1,), jnp.float32),
        ],
    )(x, batch3, w1t, b1r, w2r, b2r)

    return jnp.concatenate([attn, mean], axis=-1)
